# baseline (speedup 1.0000x reference)
import functools

import jax
import jax.numpy as jnp
from jax import lax
from jax.experimental import pallas as pl
from jax.experimental.pallas import tpu as pltpu
from jax.experimental.pallas import tpu_sc as plsc

_B = 16384
_D = 32
_NC = 2
_NS = 16
_NW = _NC * _NS
_V = 1000000
_TC = 7812
_TAIL = _TC * 128
_PERW = 248
_GA = 8
_BPW = _B // _NW
_GRP = 128
_L = 16

_mesh = plsc.VectorSubcoreMesh(
    core_axis_name="c", subcore_axis_name="s",
    num_cores=_NC, num_subcores=_NS)


@functools.partial(
    pl.kernel,
    out_type=(
        jax.ShapeDtypeStruct((4, _TC, 8, 128), jnp.float32),
        jax.ShapeDtypeStruct((4, _TC, 8, 128), jnp.float32),
    ),
    mesh=_mesh,
    compiler_params=pltpu.CompilerParams(use_tc_tiling_on_sc=True),
    scratch_types=[
        pltpu.SemaphoreType.DMA,
    ],
)
def _detile(w_sr3, w_tg3, i_sr, i_tg, sem):
  wid = lax.axis_index("s") * _NC + lax.axis_index("c")

  def body(g, _):
    cps = []
    for k in range(_GA):
      tc = jnp.minimum(wid * _PERW + g * _GA + k, _TC - 1)
      cps.append(pltpu.async_copy(
          w_sr3.at[:, :, pl.ds(tc * 128, 128)], i_sr.at[:, tc], sem))
      cps.append(pltpu.async_copy(
          w_tg3.at[:, :, pl.ds(tc * 128, 128)], i_tg.at[:, tc], sem))
    for cp in cps:
      cp.wait()
    return 0

  lax.fori_loop(0, _PERW // _GA, body, 0)


@functools.partial(
    pl.kernel,
    out_type=(
        jax.ShapeDtypeStruct((_D, _B), jnp.float32),
        jax.ShapeDtypeStruct((_D, _B), jnp.float32),
    ),
    mesh=_mesh,
    compiler_params=pltpu.CompilerParams(
        use_tc_tiling_on_sc=False, needs_layout_passes=False),
    scratch_types=[
        pltpu.VMEM((_BPW,), jnp.int32),
        pltpu.VMEM((_BPW,), jnp.int32),
        pltpu.VMEM((_L, 4, 1, 8, 8), jnp.float32),
        pltpu.VMEM((_L, 4, 1, 8, 8), jnp.float32),
        pltpu.VMEM((_D, 64), jnp.float32),
        pltpu.VMEM((_D, 64), jnp.float32),
        pltpu.VMEM((_D, _BPW), jnp.float32),
        pltpu.VMEM((_D, _BPW), jnp.float32),
        pltpu.SemaphoreType.DMA,
    ],
)
def _gather(sr_hbm, tg_hbm, i_sr, i_tg, tail_sr, tail_tg,
            out_sr_t, out_tg_t,
            idx_sr, idx_tg, blk_sr, blk_tg, tl_sr, tl_tg,
            cols_sr, cols_tg, sem):
  wid = lax.axis_index("s") * _NC + lax.axis_index("c")
  base = wid * _BPW

  pltpu.sync_copy(sr_hbm.at[pl.ds(base, _BPW)], idx_sr)
  pltpu.sync_copy(tg_hbm.at[pl.ds(base, _BPW)], idx_tg)
  pltpu.sync_copy(tail_sr, tl_sr)
  pltpu.sync_copy(tail_tg, tl_tg)

  iota = lax.iota(jnp.int32, _L)
  a_lo = iota // 8
  b_vec = lax.rem(iota, 8)

  def batch(step, _):
    i0 = step * _L
    raw_sr = idx_sr[pl.ds(i0, _L)]
    raw_tg = idx_tg[pl.ds(i0, _L)]
    v_sr = jnp.minimum(raw_sr, _TAIL - 1)
    v_tg = jnp.minimum(raw_tg, _TAIL - 1)

    cps = []
    for k in range(_L):
      j_sr = v_sr[k]
      j_tg = v_tg[k]
      l8_sr = (lax.rem(j_sr, 128) // 8) * 8
      l8_tg = (lax.rem(j_tg, 128) // 8) * 8
      cps.append(pltpu.async_copy(
          i_sr.at[:, pl.ds(j_sr // 128, 1), :, pl.ds(l8_sr, 8)],
          blk_sr.at[k], sem))
      cps.append(pltpu.async_copy(
          i_tg.at[:, pl.ds(j_tg // 128, 1), :, pl.ds(l8_tg, 8)],
          blk_tg.at[k], sem))
    for cp in cps:
      cp.wait()

    r_sr = lax.rem(v_sr, 8)
    r_tg = lax.rem(v_tg, 8)
    for k in range(_L):
      full_k = jnp.full((_L,), k, jnp.int32)
      rs = jnp.full((_L,), r_sr[k], jnp.int32)
      rt = jnp.full((_L,), r_tg[k], jnp.int32)
      col_i = jnp.full((_L,), i0 + k, jnp.int32)
      zeros = jnp.zeros((_L,), jnp.int32)
      ts_sr = jnp.full((_L,), raw_sr[k] >= _TAIL, jnp.bool_)
      ts_tg = jnp.full((_L,), raw_tg[k] >= _TAIL, jnp.bool_)
      jt_sr = jnp.full(
          (_L,), jnp.clip(raw_sr[k] - _TAIL, 0, 63), jnp.int32)
      jt_tg = jnp.full(
          (_L,), jnp.clip(raw_tg[k] - _TAIL, 0, 63), jnp.int32)
      for h in range(2):
        a_vec = a_lo + 2 * h
        c_vec = iota + h * _L
        nv_sr = plsc.load_gather(blk_sr, [full_k, a_vec, zeros, b_vec, rs])
        tv_sr = plsc.load_gather(tl_sr, [c_vec, jt_sr])
        plsc.store_scatter(
            cols_sr, [c_vec, col_i], jnp.where(ts_sr, tv_sr, nv_sr))
        nv_tg = plsc.load_gather(blk_tg, [full_k, a_vec, zeros, b_vec, rt])
        tv_tg = plsc.load_gather(tl_tg, [c_vec, jt_tg])
        plsc.store_scatter(
            cols_tg, [c_vec, col_i], jnp.where(ts_tg, tv_tg, nv_tg))
    return 0

  lax.fori_loop(0, _BPW // _L, batch, 0)


  pltpu.sync_copy(cols_sr, out_sr_t.at[:, pl.ds(base, _BPW)])
  pltpu.sync_copy(cols_tg, out_tg_t.at[:, pl.ds(base, _BPW)])


def kernel(sr_data, tg_data, W_sr, W_tg):
  w_sr3 = W_sr.T.reshape(4, 8, _V)
  w_tg3 = W_tg.T.reshape(4, 8, _V)
  i_sr, i_tg = _detile(w_sr3, w_tg3)
  tail_sr = W_sr.T[:, _TAIL:]
  tail_tg = W_tg.T[:, _TAIL:]
  out_sr_t, out_tg_t = _gather(sr_data, tg_data, i_sr, i_tg,
                               tail_sr, tail_tg)
  return (out_sr_t.T, out_tg_t.T)


# speedup vs baseline: 24.0032x; 24.0032x over previous
import functools

import jax
import jax.numpy as jnp
from jax import lax
from jax.experimental import pallas as pl
from jax.experimental.pallas import tpu as pltpu
from jax.experimental.pallas import tpu_sc as plsc

_B = 16384
_D = 32
_NC = 2
_NS = 16
_NW = _NC * _NS
_V = 1000000
_TC = 7812
_TAIL = _TC * 128
_PERW = 248
_GA = 8
_BPW = _B // _NW
_GRP = 128
_L = 16

_mesh = plsc.VectorSubcoreMesh(
    core_axis_name="c", subcore_axis_name="s",
    num_cores=_NC, num_subcores=_NS)


@functools.partial(
    pl.kernel,
    out_type=(
        jax.ShapeDtypeStruct((4, _TC, 8, 128), jnp.float32),
        jax.ShapeDtypeStruct((4, _TC, 8, 128), jnp.float32),
    ),
    mesh=_mesh,
    compiler_params=pltpu.CompilerParams(use_tc_tiling_on_sc=True),
    scratch_types=[
        pltpu.VMEM((_GA, 4, 8, 128), jnp.float32),
        pltpu.VMEM((_GA, 4, 8, 128), jnp.float32),
        pltpu.SemaphoreType.DMA,
    ],
)
def _detile(w_sr3, w_tg3, i_sr, i_tg, buf_a, buf_b, sem):
  wid = lax.axis_index("s") * _NC + lax.axis_index("c")

  def body(g, _):
    cps = []
    for k in range(_GA):
      tc = jnp.minimum(wid * _PERW + g * _GA + k, _TC - 1)
      cps.append(pltpu.async_copy(
          w_sr3.at[:, :, pl.ds(tc * 128, 128)], buf_a.at[k], sem))
      cps.append(pltpu.async_copy(
          w_tg3.at[:, :, pl.ds(tc * 128, 128)], buf_b.at[k], sem))
    for cp in cps:
      cp.wait()
    cps = []
    for k in range(_GA):
      tc = jnp.minimum(wid * _PERW + g * _GA + k, _TC - 1)
      cps.append(pltpu.async_copy(buf_a.at[k], i_sr.at[:, tc], sem))
      cps.append(pltpu.async_copy(buf_b.at[k], i_tg.at[:, tc], sem))
    for cp in cps:
      cp.wait()
    return 0

  lax.fori_loop(0, _PERW // _GA, body, 0)


@functools.partial(
    pl.kernel,
    out_type=(
        jax.ShapeDtypeStruct((_D, _B), jnp.float32),
        jax.ShapeDtypeStruct((_D, _B), jnp.float32),
    ),
    mesh=_mesh,
    compiler_params=pltpu.CompilerParams(
        use_tc_tiling_on_sc=False, needs_layout_passes=False),
    scratch_types=[
        pltpu.VMEM((_BPW,), jnp.int32),
        pltpu.VMEM((_BPW,), jnp.int32),
        pltpu.VMEM((_L, 4, 1, 8, 8), jnp.float32),
        pltpu.VMEM((_L, 4, 1, 8, 8), jnp.float32),
        pltpu.VMEM((_D, 64), jnp.float32),
        pltpu.VMEM((_D, 64), jnp.float32),
        pltpu.VMEM((_D, _BPW), jnp.float32),
        pltpu.VMEM((_D, _BPW), jnp.float32),
        pltpu.SemaphoreType.DMA,
    ],
)
def _gather(sr_hbm, tg_hbm, i_sr, i_tg, tail_sr, tail_tg,
            out_sr_t, out_tg_t,
            idx_sr, idx_tg, blk_sr, blk_tg, tl_sr, tl_tg,
            cols_sr, cols_tg, sem):
  wid = lax.axis_index("s") * _NC + lax.axis_index("c")
  base = wid * _BPW

  pltpu.sync_copy(sr_hbm.at[pl.ds(base, _BPW)], idx_sr)
  pltpu.sync_copy(tg_hbm.at[pl.ds(base, _BPW)], idx_tg)
  pltpu.sync_copy(tail_sr, tl_sr)
  pltpu.sync_copy(tail_tg, tl_tg)

  iota = lax.iota(jnp.int32, _L)
  a_lo = iota // 8
  b_vec = lax.rem(iota, 8)

  def batch(step, _):
    i0 = step * _L
    raw_sr = idx_sr[pl.ds(i0, _L)]
    raw_tg = idx_tg[pl.ds(i0, _L)]
    v_sr = jnp.minimum(raw_sr, _TAIL - 1)
    v_tg = jnp.minimum(raw_tg, _TAIL - 1)

    cps = []
    for k in range(_L):
      j_sr = v_sr[k]
      j_tg = v_tg[k]
      l8_sr = (lax.rem(j_sr, 128) // 8) * 8
      l8_tg = (lax.rem(j_tg, 128) // 8) * 8
      cps.append(pltpu.async_copy(
          i_sr.at[:, pl.ds(j_sr // 128, 1), :, pl.ds(l8_sr, 8)],
          blk_sr.at[k], sem))
      cps.append(pltpu.async_copy(
          i_tg.at[:, pl.ds(j_tg // 128, 1), :, pl.ds(l8_tg, 8)],
          blk_tg.at[k], sem))
    for cp in cps:
      cp.wait()

    r_sr = lax.rem(v_sr, 8)
    r_tg = lax.rem(v_tg, 8)
    for k in range(_L):
      full_k = jnp.full((_L,), k, jnp.int32)
      rs = jnp.full((_L,), r_sr[k], jnp.int32)
      rt = jnp.full((_L,), r_tg[k], jnp.int32)
      col_i = jnp.full((_L,), i0 + k, jnp.int32)
      zeros = jnp.zeros((_L,), jnp.int32)
      ts_sr = jnp.full((_L,), raw_sr[k] >= _TAIL, jnp.bool_)
      ts_tg = jnp.full((_L,), raw_tg[k] >= _TAIL, jnp.bool_)
      jt_sr = jnp.full(
          (_L,), jnp.clip(raw_sr[k] - _TAIL, 0, 63), jnp.int32)
      jt_tg = jnp.full(
          (_L,), jnp.clip(raw_tg[k] - _TAIL, 0, 63), jnp.int32)
      for h in range(2):
        a_vec = a_lo + 2 * h
        c_vec = iota + h * _L
        nv_sr = plsc.load_gather(blk_sr, [full_k, a_vec, zeros, b_vec, rs])
        tv_sr = plsc.load_gather(tl_sr, [c_vec, jt_sr])
        plsc.store_scatter(
            cols_sr, [c_vec, col_i], jnp.where(ts_sr, tv_sr, nv_sr))
        nv_tg = plsc.load_gather(blk_tg, [full_k, a_vec, zeros, b_vec, rt])
        tv_tg = plsc.load_gather(tl_tg, [c_vec, jt_tg])
        plsc.store_scatter(
            cols_tg, [c_vec, col_i], jnp.where(ts_tg, tv_tg, nv_tg))
    return 0

  lax.fori_loop(0, _BPW // _L, batch, 0)


  pltpu.sync_copy(cols_sr, out_sr_t.at[:, pl.ds(base, _BPW)])
  pltpu.sync_copy(cols_tg, out_tg_t.at[:, pl.ds(base, _BPW)])


def kernel(sr_data, tg_data, W_sr, W_tg):
  w_sr3 = W_sr.T.reshape(4, 8, _V)
  w_tg3 = W_tg.T.reshape(4, 8, _V)
  i_sr, i_tg = _detile(w_sr3, w_tg3)
  tail_sr = W_sr.T[:, _TAIL:]
  tail_tg = W_tg.T[:, _TAIL:]
  out_sr_t, out_tg_t = _gather(sr_data, tg_data, i_sr, i_tg,
                               tail_sr, tail_tg)
  return (out_sr_t.T, out_tg_t.T)
